# Initial kernel scaffold; baseline (speedup 1.0000x reference)
import jax
import jax.numpy as jnp
from jax.experimental import pallas as pl


def kernel(x, edge_index, W1, b1, W2, b2, Wlin, blin):
    raise NotImplementedError("write your pallas kernel here")


# speedup vs baseline: 14.4311x; 14.4311x over previous
import functools

import jax
import jax.numpy as jnp
from jax import lax
from jax.experimental import pallas as pl
from jax.experimental.pallas import tpu as pltpu
from jax.experimental.pallas import tpu_sc as plsc

N = 10000
NPAD = 10240
E = 320000
IN = 128
HID = 256
OUT = 128

NC = 2
NS = 16
EPC = E // NC
EPW = EPC // NS
C = 80
NCHUNK = EPW // C
RPT = NPAD // NS
WB = 80
NWB = RPT // WB

_mesh = plsc.VectorSubcoreMesh(core_axis_name="c", subcore_axis_name="s")


@functools.partial(
    pl.kernel,
    out_type=jax.ShapeDtypeStruct((NC, NPAD), jnp.float32),
    mesh=_mesh,
    scratch_types=[
        pltpu.VMEM((C,), jnp.int32),
        pltpu.VMEM((C,), jnp.float32),
        pltpu.VMEM((RPT,), jnp.float32),
        pltpu.VMEM_SHARED((NPAD,), jnp.float32),
    ],
)
def _deg_kernel(dst_hbm, ones_hbm, zeros_hbm, out_hbm, idx_v, ones_v, stage_v, acc):
    c = lax.axis_index("c")
    s = lax.axis_index("s")
    pltpu.sync_copy(ones_hbm, ones_v)
    pltpu.sync_copy(zeros_hbm, stage_v)
    pltpu.sync_copy(stage_v, acc.at[pl.ds(s * RPT, RPT)])
    plsc.subcore_barrier()

    ebase = c * EPC + s * EPW

    def body(i, carry):
        base = pl.multiple_of(ebase + i * C, 8)
        pltpu.sync_copy(dst_hbm.at[pl.ds(base, C)], idx_v)
        pltpu.sync_copy(ones_v, acc.at[idx_v], add=True)
        return carry

    lax.fori_loop(0, NCHUNK, body, 0)
    plsc.subcore_barrier()
    pltpu.sync_copy(acc.at[pl.ds(s * RPT, RPT)], stage_v)
    pltpu.sync_copy(stage_v, out_hbm.at[c, pl.ds(s * RPT, RPT)])


@functools.partial(
    pl.kernel,
    out_type=jax.ShapeDtypeStruct((NC, NPAD, IN), jnp.float32),
    mesh=_mesh,
    scratch_types=[
        pltpu.VMEM((C,), jnp.int32),
        pltpu.VMEM((C,), jnp.int32),
        pltpu.VMEM((C, IN), jnp.float32),
        pltpu.VMEM_SHARED((NPAD, IN), jnp.float32),
        pltpu.SemaphoreType.DMA,
    ],
)
def _agg_kernel(src_hbm, dst_hbm, tab_hbm, zrows_hbm, out_hbm,
                src_v, dst_v, rows_v, acc, sem):
    c = lax.axis_index("c")
    s = lax.axis_index("s")

    pltpu.sync_copy(zrows_hbm, rows_v)
    for k in range(NWB):
        pltpu.sync_copy(rows_v, acc.at[pl.ds(s * RPT + k * WB, WB)])
    plsc.subcore_barrier()

    ebase = c * EPC + s * EPW

    def body(i, carry):
        base = pl.multiple_of(ebase + i * C, 8)
        pltpu.sync_copy(src_hbm.at[pl.ds(base, C)], src_v)
        pltpu.sync_copy(dst_hbm.at[pl.ds(base, C)], dst_v)
        pltpu.async_copy(tab_hbm.at[src_v], rows_v, sem).wait()
        pltpu.sync_copy(rows_v, acc.at[dst_v], add=True)
        return carry

    lax.fori_loop(0, NCHUNK, body, 0)
    plsc.subcore_barrier()
    for k in range(NWB):
        r0 = s * RPT + k * WB
        pltpu.sync_copy(acc.at[pl.ds(r0, WB)], rows_v)
        pltpu.sync_copy(rows_v, out_hbm.at[c, pl.ds(r0, WB)])


RB = 1024
GRID = NPAD // RB


def _dis_block(pt):
    return lax.rsqrt(1.0 + pt[:, 0:1] + pt[:, 1:2])


def _xs_body(pt_ref, x_ref, xs_ref):
    dis = _dis_block(pt_ref[...])
    xs_ref[...] = x_ref[...] * dis


def _mid_body(pt_ref, x_ref, s1_ref, W1_ref, b1_ref, W2_ref, y_ref, ys_ref):
    dis = _dis_block(pt_ref[...])
    agg = s1_ref[0] + s1_ref[1]
    u = dis * agg + (dis * dis) * x_ref[...]
    h = jnp.dot(u, W1_ref[...], preferred_element_type=jnp.float32) + b1_ref[...]
    h = jnp.maximum(h, 0.0)
    y = jnp.dot(h, W2_ref[...], preferred_element_type=jnp.float32)
    y_ref[...] = y
    ys_ref[...] = y * dis


def _out_body(pt_ref, y_ref, s2_ref, b2_ref, Wl_ref, bl_ref, o_ref):
    dis = _dis_block(pt_ref[...])
    agg = s2_ref[0] + s2_ref[1]
    h2 = jnp.maximum(dis * agg + (dis * dis) * y_ref[...] + b2_ref[...], 0.0)
    o_ref[...] = jnp.dot(h2, Wl_ref[...], preferred_element_type=jnp.float32) + bl_ref[...]


def _row_spec(width):
    return pl.BlockSpec((RB, width), lambda i: (i, 0))


def _part_spec(width):
    return pl.BlockSpec((NC, RB, width), lambda i: (0, i, 0))


def _full_spec(shape):
    return pl.BlockSpec(shape, lambda i: tuple(0 for _ in shape))


_xs_call = pl.pallas_call(
    _xs_body,
    grid=(GRID,),
    in_specs=[_row_spec(2), _row_spec(IN)],
    out_specs=_row_spec(IN),
    out_shape=jax.ShapeDtypeStruct((NPAD, IN), jnp.float32),
)

_mid_call = pl.pallas_call(
    _mid_body,
    grid=(GRID,),
    in_specs=[
        _row_spec(2),
        _row_spec(IN),
        _part_spec(IN),
        _full_spec((IN, HID)),
        _full_spec((1, HID)),
        _full_spec((HID, OUT)),
    ],
    out_specs=[_row_spec(OUT), _row_spec(OUT)],
    out_shape=[
        jax.ShapeDtypeStruct((NPAD, OUT), jnp.float32),
        jax.ShapeDtypeStruct((NPAD, OUT), jnp.float32),
    ],
)

_out_call = pl.pallas_call(
    _out_body,
    grid=(GRID,),
    in_specs=[
        _row_spec(2),
        _row_spec(OUT),
        _part_spec(OUT),
        _full_spec((1, OUT)),
        _full_spec((OUT, 128)),
        _full_spec((1, 128)),
    ],
    out_specs=_row_spec(128),
    out_shape=jax.ShapeDtypeStruct((NPAD, 128), jnp.float32),
)


def kernel(x, edge_index, W1, b1, W2, b2, Wlin, blin):
    f32 = jnp.float32
    src = edge_index[0]
    dst = edge_index[1]
    xpad = jnp.pad(x, ((0, NPAD - N), (0, 0)))
    ones_c = jnp.ones((C,), f32)
    zeros_1d = jnp.zeros((RPT,), f32)
    zeros_rows = jnp.zeros((C, IN), f32)

    degp = _deg_kernel(dst, ones_c, zeros_1d)
    pt = degp.T
    xs = _xs_call(pt, xpad)
    s1 = _agg_kernel(src, dst, xs, zeros_rows)
    y, ys = _mid_call(pt, xpad, s1, W1, b1.reshape(1, HID), W2)
    s2 = _agg_kernel(src, dst, ys, zeros_rows)
    Wl = jnp.zeros((OUT, 128), f32).at[:, :2].set(Wlin)
    bl = jnp.zeros((1, 128), f32).at[0, :2].set(blin)
    o = _out_call(pt, y, s2, b2.reshape(1, OUT), Wl, bl)
    return o[:N, :2]


# speedup vs baseline: 27.0378x; 1.8736x over previous
import functools

import jax
import jax.numpy as jnp
from jax import lax
from jax.experimental import pallas as pl
from jax.experimental.pallas import tpu as pltpu
from jax.experimental.pallas import tpu_sc as plsc

N = 10000
NPAD = 10240
E = 320000
IN = 128
HID = 256
OUT = 128

NC = 2
NS = 16
EPC = E // NC
EPW = EPC // NS
C = 80
NCHUNK = EPW // C
RPT = NPAD // NS
WB = 80
NWB = RPT // WB

_mesh = plsc.VectorSubcoreMesh(core_axis_name="c", subcore_axis_name="s")


@functools.partial(
    pl.kernel,
    out_type=jax.ShapeDtypeStruct((NC, NPAD), jnp.float32),
    mesh=_mesh,
    scratch_types=[
        pltpu.VMEM((C,), jnp.int32),
        pltpu.VMEM((C,), jnp.int32),
        pltpu.VMEM((C,), jnp.float32),
        pltpu.VMEM((RPT,), jnp.float32),
        pltpu.VMEM_SHARED((NPAD,), jnp.float32),
        pltpu.SemaphoreType.DMA,
        pltpu.SemaphoreType.DMA,
    ],
)
def _deg_kernel(dst_hbm, ones_hbm, zeros_hbm, out_hbm,
                idx0, idx1, ones_v, stage_v, acc, sem0, sem1):
    c = lax.axis_index("c")
    s = lax.axis_index("s")
    pltpu.sync_copy(ones_hbm, ones_v)
    pltpu.sync_copy(zeros_hbm, stage_v)
    pltpu.sync_copy(stage_v, acc.at[pl.ds(s * RPT, RPT)])
    plsc.subcore_barrier()

    ebase = c * EPC + s * EPW
    idx = (idx0, idx1)
    sem = (sem0, sem1)

    def src_at(j):
        return dst_hbm.at[pl.ds(pl.multiple_of(ebase + j * C, 8), C)]

    def load(j, b):
        pltpu.async_copy(src_at(j), idx[b], sem[b])

    def wait_scatter(j, b):
        pltpu.make_async_copy(src_at(j), idx[b], sem[b]).wait()
        pltpu.sync_copy(ones_v, acc.at[idx[b]], add=True)

    load(0, 0)
    load(1, 1)

    def body(k, carry):
        j = 2 * k
        wait_scatter(j, 0)
        load(j + 2, 0)
        wait_scatter(j + 1, 1)
        load(j + 3, 1)
        return carry

    KMAIN = (NCHUNK - 3) // 2
    lax.fori_loop(0, KMAIN, body, 0)
    for j in range(2 * KMAIN, NCHUNK):
        wait_scatter(j, j % 2)
        if j + 2 < NCHUNK:
            load(j + 2, j % 2)

    plsc.subcore_barrier()
    pltpu.sync_copy(acc.at[pl.ds(s * RPT, RPT)], stage_v)
    pltpu.sync_copy(stage_v, out_hbm.at[c, pl.ds(s * RPT, RPT)])


@functools.partial(
    pl.kernel,
    out_type=jax.ShapeDtypeStruct((NC, NPAD, IN), jnp.float32),
    mesh=_mesh,
    scratch_types=[
        pltpu.VMEM((C,), jnp.int32),
        pltpu.VMEM((C,), jnp.int32),
        pltpu.VMEM((C,), jnp.int32),
        pltpu.VMEM((C,), jnp.int32),
        pltpu.VMEM((C, IN), jnp.float32),
        pltpu.VMEM((C, IN), jnp.float32),
        pltpu.VMEM_SHARED((NPAD, IN), jnp.float32),
        pltpu.SemaphoreType.DMA,
        pltpu.SemaphoreType.DMA,
        pltpu.SemaphoreType.DMA,
        pltpu.SemaphoreType.DMA,
    ],
)
def _agg_kernel(src_hbm, dst_hbm, tab_hbm, zrows_hbm, out_hbm,
                src0, dst0, src1, dst1, rows0, rows1, acc,
                semi0, semi1, semg0, semg1):
    c = lax.axis_index("c")
    s = lax.axis_index("s")

    pltpu.sync_copy(zrows_hbm, rows0)
    for k in range(NWB):
        pltpu.sync_copy(rows0, acc.at[pl.ds(s * RPT + k * WB, WB)])
    plsc.subcore_barrier()

    ebase = c * EPC + s * EPW
    srcv = (src0, src1)
    dstv = (dst0, dst1)
    rows = (rows0, rows1)
    semi = (semi0, semi1)
    semg = (semg0, semg1)

    def s_at(j):
        return src_hbm.at[pl.ds(pl.multiple_of(ebase + j * C, 8), C)]

    def d_at(j):
        return dst_hbm.at[pl.ds(pl.multiple_of(ebase + j * C, 8), C)]

    def load_idx(j, b):
        pltpu.async_copy(s_at(j), srcv[b], semi[b])
        pltpu.async_copy(d_at(j), dstv[b], semi[b])

    def start_gather(j, b):
        pltpu.make_async_copy(s_at(j), srcv[b], semi[b]).wait()
        pltpu.make_async_copy(d_at(j), dstv[b], semi[b]).wait()
        pltpu.async_copy(tab_hbm.at[srcv[b]], rows[b], semg[b])

    def wait_scatter(b):
        pltpu.make_async_copy(tab_hbm.at[srcv[b]], rows[b], semg[b]).wait()
        pltpu.sync_copy(rows[b], acc.at[dstv[b]], add=True)

    load_idx(0, 0)
    start_gather(0, 0)
    load_idx(1, 1)

    def body(k, carry):
        j = 2 * k
        start_gather(j + 1, 1)
        wait_scatter(0)
        load_idx(j + 2, 0)
        start_gather(j + 2, 0)
        wait_scatter(1)
        load_idx(j + 3, 1)
        return carry

    KMAIN = (NCHUNK - 3) // 2
    lax.fori_loop(0, KMAIN, body, 0)
    j0 = 2 * KMAIN
    for j in range(j0 + 1, NCHUNK):
        start_gather(j, j % 2)
        wait_scatter((j - 1) % 2)
        if j + 1 < NCHUNK:
            load_idx(j + 1, (j - 1) % 2)
    wait_scatter((NCHUNK - 1) % 2)

    plsc.subcore_barrier()
    for k in range(NWB):
        r0 = s * RPT + k * WB
        pltpu.sync_copy(acc.at[pl.ds(r0, WB)], rows0)
        pltpu.sync_copy(rows0, out_hbm.at[c, pl.ds(r0, WB)])


RB = 1024
GRID = NPAD // RB


def _dis_block(pt):
    return lax.rsqrt(1.0 + pt[:, 0:1] + pt[:, 1:2])


def _xs_body(pt_ref, x_ref, xs_ref):
    dis = _dis_block(pt_ref[...])
    xs_ref[...] = x_ref[...] * dis


def _mid_body(pt_ref, x_ref, s1_ref, W1_ref, b1_ref, W2_ref, y_ref, ys_ref):
    dis = _dis_block(pt_ref[...])
    agg = s1_ref[0] + s1_ref[1]
    u = dis * agg + (dis * dis) * x_ref[...]
    h = jnp.dot(u, W1_ref[...], preferred_element_type=jnp.float32) + b1_ref[...]
    h = jnp.maximum(h, 0.0)
    y = jnp.dot(h, W2_ref[...], preferred_element_type=jnp.float32)
    y_ref[...] = y
    ys_ref[...] = y * dis


def _out_body(pt_ref, y_ref, s2_ref, b2_ref, Wl_ref, bl_ref, o_ref):
    dis = _dis_block(pt_ref[...])
    agg = s2_ref[0] + s2_ref[1]
    h2 = jnp.maximum(dis * agg + (dis * dis) * y_ref[...] + b2_ref[...], 0.0)
    o_ref[...] = jnp.dot(h2, Wl_ref[...], preferred_element_type=jnp.float32) + bl_ref[...]


def _row_spec(width):
    return pl.BlockSpec((RB, width), lambda i: (i, 0))


def _part_spec(width):
    return pl.BlockSpec((NC, RB, width), lambda i: (0, i, 0))


def _full_spec(shape):
    return pl.BlockSpec(shape, lambda i: tuple(0 for _ in shape))


_xs_call = pl.pallas_call(
    _xs_body,
    grid=(GRID,),
    in_specs=[_row_spec(2), _row_spec(IN)],
    out_specs=_row_spec(IN),
    out_shape=jax.ShapeDtypeStruct((NPAD, IN), jnp.float32),
)

_mid_call = pl.pallas_call(
    _mid_body,
    grid=(GRID,),
    in_specs=[
        _row_spec(2),
        _row_spec(IN),
        _part_spec(IN),
        _full_spec((IN, HID)),
        _full_spec((1, HID)),
        _full_spec((HID, OUT)),
    ],
    out_specs=[_row_spec(OUT), _row_spec(OUT)],
    out_shape=[
        jax.ShapeDtypeStruct((NPAD, OUT), jnp.float32),
        jax.ShapeDtypeStruct((NPAD, OUT), jnp.float32),
    ],
)

_out_call = pl.pallas_call(
    _out_body,
    grid=(GRID,),
    in_specs=[
        _row_spec(2),
        _row_spec(OUT),
        _part_spec(OUT),
        _full_spec((1, OUT)),
        _full_spec((OUT, 128)),
        _full_spec((1, 128)),
    ],
    out_specs=_row_spec(128),
    out_shape=jax.ShapeDtypeStruct((NPAD, 128), jnp.float32),
)


def kernel(x, edge_index, W1, b1, W2, b2, Wlin, blin):
    f32 = jnp.float32
    src = edge_index[0]
    dst = edge_index[1]
    xpad = jnp.pad(x, ((0, NPAD - N), (0, 0)))
    ones_c = jnp.ones((C,), f32)
    zeros_1d = jnp.zeros((RPT,), f32)
    zeros_rows = jnp.zeros((C, IN), f32)

    degp = _deg_kernel(dst, ones_c, zeros_1d)
    pt = degp.T
    xs = _xs_call(pt, xpad)
    s1 = _agg_kernel(src, dst, xs, zeros_rows)
    y, ys = _mid_call(pt, xpad, s1, W1, b1.reshape(1, HID), W2)
    s2 = _agg_kernel(src, dst, ys, zeros_rows)
    Wl = jnp.zeros((OUT, 128), f32).at[:, :2].set(Wlin)
    bl = jnp.zeros((1, 128), f32).at[0, :2].set(blin)
    o = _out_call(pt, y, s2, b2.reshape(1, OUT), Wl, bl)
    return o[:N, :2]


# speedup vs baseline: 36.1912x; 1.3385x over previous
import functools

import jax
import jax.numpy as jnp
from jax import lax
from jax.experimental import pallas as pl
from jax.experimental.pallas import tpu as pltpu
from jax.experimental.pallas import tpu_sc as plsc

N = 10000
NPAD = 10240
E = 320000
IN = 128
HID = 256
OUT = 128

NC = 2
NS = 16
EPC = E // NC
EPW = EPC // NS
C = 80
NCHUNK = EPW // C
RPT = NPAD // NS
WB = 80
NWB = RPT // WB

_mesh = plsc.VectorSubcoreMesh(core_axis_name="c", subcore_axis_name="s")


@functools.partial(
    pl.kernel,
    out_type=jax.ShapeDtypeStruct((NC, NPAD), jnp.float32),
    mesh=_mesh,
    scratch_types=[
        [pltpu.VMEM((C,), jnp.int32)] * 4,
        pltpu.VMEM((C,), jnp.float32),
        pltpu.VMEM((RPT,), jnp.float32),
        pltpu.VMEM_SHARED((NPAD,), jnp.float32),
        [pltpu.SemaphoreType.DMA] * 4,
        [pltpu.SemaphoreType.DMA] * 4,
    ],
)
def _deg_kernel(dst_hbm, ones_hbm, zeros_hbm, out_hbm,
                idx, ones_v, stage_v, acc, semi, sems):
    c = lax.axis_index("c")
    s = lax.axis_index("s")
    pltpu.sync_copy(ones_hbm, ones_v)
    pltpu.sync_copy(zeros_hbm, stage_v)
    pltpu.sync_copy(stage_v, acc.at[pl.ds(s * RPT, RPT)])
    plsc.subcore_barrier()

    ebase = c * EPC + s * EPW

    def src_at(j):
        return dst_hbm.at[pl.ds(pl.multiple_of(ebase + j * C, 8), C)]

    def load(j, b):
        pltpu.async_copy(src_at(j), idx[b], semi[b])

    def wait_scatter(b):
        pltpu.make_async_copy(ones_v, acc.at[idx[b]], sems[b]).wait()

    def step(j, b, do_load, do_wait):
        if do_load:
            if do_wait:
                wait_scatter((b + 2) % 4)
            load(j + 2, (b + 2) % 4)
        pltpu.make_async_copy(src_at(j), idx[b], semi[b]).wait()
        pltpu.async_copy(ones_v, acc.at[idx[b]], sems[b], add=True)

    load(0, 0)
    load(1, 1)

    def body(k, carry):
        for r in range(4):
            step(4 * k + 2 + r, (2 + r) % 4, True, True)
        return carry

    KMAIN = (NCHUNK - 4) // 4
    step(0, 0, True, False)
    step(1, 1, True, False)
    lax.fori_loop(0, KMAIN, body, 0)
    for j in range(4 * KMAIN + 2, NCHUNK):
        step(j, j % 4, j + 2 < NCHUNK, j - 2 >= 0)
    for j in range(max(0, NCHUNK - 4), NCHUNK):
        wait_scatter(j % 4)

    plsc.subcore_barrier()
    pltpu.sync_copy(acc.at[pl.ds(s * RPT, RPT)], stage_v)
    pltpu.sync_copy(stage_v, out_hbm.at[c, pl.ds(s * RPT, RPT)])


@functools.partial(
    pl.kernel,
    out_type=jax.ShapeDtypeStruct((NC, NPAD, IN), jnp.float32),
    mesh=_mesh,
    scratch_types=[
        [pltpu.VMEM((C,), jnp.int32)] * 4,
        [pltpu.VMEM((C,), jnp.int32)] * 4,
        [pltpu.VMEM((C, IN), jnp.float32)] * 4,
        pltpu.VMEM_SHARED((NPAD, IN), jnp.float32),
        [pltpu.SemaphoreType.DMA] * 4,
        [pltpu.SemaphoreType.DMA] * 4,
        [pltpu.SemaphoreType.DMA] * 4,
    ],
)
def _agg_kernel(src_hbm, dst_hbm, tab_hbm, zrows_hbm, out_hbm,
                srcv, dstv, rows, acc, semi, semg, sems):
    c = lax.axis_index("c")
    s = lax.axis_index("s")

    pltpu.sync_copy(zrows_hbm, rows[0])
    for k in range(NWB):
        pltpu.sync_copy(rows[0], acc.at[pl.ds(s * RPT + k * WB, WB)])
    plsc.subcore_barrier()

    ebase = c * EPC + s * EPW

    def s_at(j):
        return src_hbm.at[pl.ds(pl.multiple_of(ebase + j * C, 8), C)]

    def d_at(j):
        return dst_hbm.at[pl.ds(pl.multiple_of(ebase + j * C, 8), C)]

    def load_idx(j, b):
        pltpu.async_copy(s_at(j), srcv[b], semi[b])
        pltpu.async_copy(d_at(j), dstv[b], semi[b])

    def start_gather(j, b):
        pltpu.make_async_copy(s_at(j), srcv[b], semi[b]).wait()
        pltpu.make_async_copy(d_at(j), dstv[b], semi[b]).wait()
        pltpu.async_copy(tab_hbm.at[srcv[b]], rows[b], semg[b])

    def start_scatter(b):
        pltpu.make_async_copy(tab_hbm.at[srcv[b]], rows[b], semg[b]).wait()
        pltpu.async_copy(rows[b], acc.at[dstv[b]], sems[b], add=True)

    def wait_scatter(b):
        pltpu.make_async_copy(rows[b], acc.at[dstv[b]], sems[b]).wait()

    def step(j, b, do_load, do_wait, do_gather):
        if do_load:
            if do_wait:
                wait_scatter((b + 2) % 4)
            load_idx(j + 2, (b + 2) % 4)
        if do_gather:
            start_gather(j + 1, (b + 1) % 4)
        start_scatter(b)

    load_idx(0, 0)
    load_idx(1, 1)
    start_gather(0, 0)

    def body(k, carry):
        for r in range(4):
            step(4 * k + 2 + r, (2 + r) % 4, True, True, True)
        return carry

    KMAIN = (NCHUNK - 4) // 4
    step(0, 0, True, False, True)
    step(1, 1, True, False, True)
    lax.fori_loop(0, KMAIN, body, 0)
    for j in range(4 * KMAIN + 2, NCHUNK):
        step(j, j % 4, j + 2 < NCHUNK, j - 2 >= 0, j + 1 < NCHUNK)
    for j in range(max(0, NCHUNK - 4), NCHUNK):
        wait_scatter(j % 4)

    plsc.subcore_barrier()
    for k in range(NWB):
        r0 = s * RPT + k * WB
        pltpu.sync_copy(acc.at[pl.ds(r0, WB)], rows[0])
        pltpu.sync_copy(rows[0], out_hbm.at[c, pl.ds(r0, WB)])


RB = 1024
GRID = NPAD // RB


def _dis_block(pt):
    return lax.rsqrt(1.0 + pt[:, 0:1] + pt[:, 1:2])


def _xs_body(pt_ref, x_ref, xs_ref):
    dis = _dis_block(pt_ref[...])
    xs_ref[...] = x_ref[...] * dis


def _mid_body(pt_ref, x_ref, s1_ref, W1_ref, b1_ref, W2_ref, y_ref, ys_ref):
    dis = _dis_block(pt_ref[...])
    agg = s1_ref[0] + s1_ref[1]
    u = dis * agg + (dis * dis) * x_ref[...]
    h = jnp.dot(u, W1_ref[...], preferred_element_type=jnp.float32) + b1_ref[...]
    h = jnp.maximum(h, 0.0)
    y = jnp.dot(h, W2_ref[...], preferred_element_type=jnp.float32)
    y_ref[...] = y
    ys_ref[...] = y * dis


def _out_body(pt_ref, y_ref, s2_ref, b2_ref, Wl_ref, bl_ref, o_ref):
    dis = _dis_block(pt_ref[...])
    agg = s2_ref[0] + s2_ref[1]
    h2 = jnp.maximum(dis * agg + (dis * dis) * y_ref[...] + b2_ref[...], 0.0)
    o_ref[...] = jnp.dot(h2, Wl_ref[...], preferred_element_type=jnp.float32) + bl_ref[...]


def _row_spec(width):
    return pl.BlockSpec((RB, width), lambda i: (i, 0))


def _part_spec(width):
    return pl.BlockSpec((NC, RB, width), lambda i: (0, i, 0))


def _full_spec(shape):
    return pl.BlockSpec(shape, lambda i: tuple(0 for _ in shape))


_xs_call = pl.pallas_call(
    _xs_body,
    grid=(GRID,),
    in_specs=[_row_spec(2), _row_spec(IN)],
    out_specs=_row_spec(IN),
    out_shape=jax.ShapeDtypeStruct((NPAD, IN), jnp.float32),
)

_mid_call = pl.pallas_call(
    _mid_body,
    grid=(GRID,),
    in_specs=[
        _row_spec(2),
        _row_spec(IN),
        _part_spec(IN),
        _full_spec((IN, HID)),
        _full_spec((1, HID)),
        _full_spec((HID, OUT)),
    ],
    out_specs=[_row_spec(OUT), _row_spec(OUT)],
    out_shape=[
        jax.ShapeDtypeStruct((NPAD, OUT), jnp.float32),
        jax.ShapeDtypeStruct((NPAD, OUT), jnp.float32),
    ],
)

_out_call = pl.pallas_call(
    _out_body,
    grid=(GRID,),
    in_specs=[
        _row_spec(2),
        _row_spec(OUT),
        _part_spec(OUT),
        _full_spec((1, OUT)),
        _full_spec((OUT, 128)),
        _full_spec((1, 128)),
    ],
    out_specs=_row_spec(128),
    out_shape=jax.ShapeDtypeStruct((NPAD, 128), jnp.float32),
)


def kernel(x, edge_index, W1, b1, W2, b2, Wlin, blin):
    f32 = jnp.float32
    src = edge_index[0]
    dst = edge_index[1]
    xpad = jnp.pad(x, ((0, NPAD - N), (0, 0)))
    ones_c = jnp.ones((C,), f32)
    zeros_1d = jnp.zeros((RPT,), f32)
    zeros_rows = jnp.zeros((C, IN), f32)

    degp = _deg_kernel(dst, ones_c, zeros_1d)
    pt = degp.T
    xs = _xs_call(pt, xpad)
    s1 = _agg_kernel(src, dst, xs, zeros_rows)
    y, ys = _mid_call(pt, xpad, s1, W1, b1.reshape(1, HID), W2)
    s2 = _agg_kernel(src, dst, ys, zeros_rows)
    Wl = jnp.zeros((OUT, 128), f32).at[:, :2].set(Wlin)
    bl = jnp.zeros((1, 128), f32).at[0, :2].set(blin)
    o = _out_call(pt, y, s2, b2.reshape(1, OUT), Wl, bl)
    return o[:N, :2]
